# baseline (speedup 1.0000x reference)
import functools

import jax
import jax.numpy as jnp
from jax import lax
from jax.experimental import pallas as pl
from jax.experimental.pallas import tpu as pltpu
from jax.experimental.pallas import tpu_sc as plsc

_NC = 2
_NS = 16
_NW = _NC * _NS
_CHUNK = 128
_OUT_CHUNK = 400


def _sc_gather(ps, pd, src, dst, e0, ecount):
    D = ps.shape[1]
    dt = ps.dtype
    n_chunks = ecount // _CHUNK
    iters = (n_chunks + _NW - 1) // _NW

    mesh = plsc.VectorSubcoreMesh(core_axis_name="c", subcore_axis_name="s")

    pairs = (iters + 1) // 2

    def body(ps_hbm, pd_hbm, src_hbm, dst_hbm, gs_hbm, gd_hbm,
             idx_s0, idx_d0, buf_s0, buf_d0, idx_s1, idx_d1, buf_s1, buf_d1,
             sem_i0, sem_i1, sem_g0, sem_g1, sem_w0, sem_w1):
        idx_s = (idx_s0, idx_s1)
        idx_d = (idx_d0, idx_d1)
        buf_s = (buf_s0, buf_s1)
        buf_d = (buf_d0, buf_d1)
        sem_i = (sem_i0, sem_i1)
        sem_g = (sem_g0, sem_g1)
        sem_w = (sem_w0, sem_w1)
        wid = lax.axis_index("s") * _NC + lax.axis_index("c")

        def base_of(j):
            return (wid + j * _NW) * _CHUNK

        def valid(j):
            return wid + j * _NW < n_chunks

        def issue_idx(j, b):
            pltpu.async_copy(src_hbm.at[pl.ds(e0 + base_of(j), _CHUNK)],
                             idx_s[b], sem_i[b])
            pltpu.async_copy(dst_hbm.at[pl.ds(e0 + base_of(j), _CHUNK)],
                             idx_d[b], sem_i[b])

        def wait_idx(j, b):
            pltpu.make_async_copy(src_hbm.at[pl.ds(e0 + base_of(j), _CHUNK)],
                                  idx_s[b], sem_i[b]).wait()
            pltpu.make_async_copy(dst_hbm.at[pl.ds(e0 + base_of(j), _CHUNK)],
                                  idx_d[b], sem_i[b]).wait()

        def wait_wb(j, b):
            pltpu.make_async_copy(
                buf_s[b], gs_hbm.at[pl.ds(base_of(j), _CHUNK)],
                sem_w[b]).wait()
            pltpu.make_async_copy(
                buf_d[b], gd_hbm.at[pl.ds(base_of(j), _CHUNK)],
                sem_w[b]).wait()

        @pl.when(valid(0))
        def _():
            issue_idx(0, 0)

        def pair_step(i, _):
            for b in (0, 1):
                j = 2 * i + b

                @pl.when((j >= 2) & valid(j))
                def _():
                    wait_wb(j, b)

                @pl.when(valid(j))
                def _():
                    wait_idx(j, b)
                    pltpu.async_copy(ps_hbm.at[idx_s[b]], buf_s[b], sem_g[b])
                    pltpu.async_copy(pd_hbm.at[idx_d[b]], buf_d[b], sem_g[b])

                @pl.when(valid(j + 1))
                def _():
                    issue_idx(j + 1, 1 - b)

                @pl.when(valid(j))
                def _():
                    pltpu.make_async_copy(ps_hbm.at[idx_s[b]], buf_s[b],
                                          sem_g[b]).wait()
                    pltpu.make_async_copy(pd_hbm.at[idx_d[b]], buf_d[b],
                                          sem_g[b]).wait()
                    pltpu.async_copy(buf_s[b],
                                     gs_hbm.at[pl.ds(base_of(j), _CHUNK)],
                                     sem_w[b])
                    pltpu.async_copy(buf_d[b],
                                     gd_hbm.at[pl.ds(base_of(j), _CHUNK)],
                                     sem_w[b])

            return _

        lax.fori_loop(0, pairs, pair_step, None)
        for jt in (2 * pairs - 2, 2 * pairs - 1):
            if jt >= 0:
                bt = jt % 2

                @pl.when(valid(jt))
                def _():
                    wait_wb(jt, bt)

    scratch = [
        pltpu.VMEM((_CHUNK,), jnp.int32), pltpu.VMEM((_CHUNK,), jnp.int32),
        pltpu.VMEM((_CHUNK, D), dt), pltpu.VMEM((_CHUNK, D), dt),
        pltpu.VMEM((_CHUNK,), jnp.int32), pltpu.VMEM((_CHUNK,), jnp.int32),
        pltpu.VMEM((_CHUNK, D), dt), pltpu.VMEM((_CHUNK, D), dt),
    ] + [pltpu.SemaphoreType.DMA] * 6

    return pl.kernel(
        body,
        mesh=mesh,
        out_type=(
            jax.ShapeDtypeStruct((ecount, D), dt),
            jax.ShapeDtypeStruct((ecount, D), dt),
        ),
        scratch_types=scratch,
    )(ps, pd, src, dst)


def _sc_segment_sum(e_split, dst, init, e0, num_nodes):
    _, E, DH = e_split.shape
    n_chunks = E // _CHUNK
    iters = (n_chunks + _NS - 1) // _NS
    n_out_chunks = num_nodes // _OUT_CHUNK
    out_iters = (n_out_chunks + _NS - 1) // _NS

    pairs = (iters + 1) // 2
    mesh = plsc.VectorSubcoreMesh(core_axis_name="c", subcore_axis_name="s")

    @functools.partial(
        pl.kernel,
        mesh=mesh,
        out_type=jax.ShapeDtypeStruct((_NC, num_nodes, DH), jnp.float32),
        scratch_types=[
            pltpu.VMEM((_CHUNK,), jnp.int32),
            pltpu.VMEM((_CHUNK, DH), jnp.float32),
            pltpu.VMEM((_CHUNK,), jnp.int32),
            pltpu.VMEM((_CHUNK, DH), jnp.float32),
            pltpu.VMEM_SHARED((num_nodes, DH), jnp.float32),
            pltpu.SemaphoreType.DMA,
            pltpu.SemaphoreType.DMA,
        ],
    )
    def body(e_hbm, dst_hbm, z_hbm, agg_hbm, idx0, rows0, idx1, rows1,
             acc_sh, sem0, sem1):
        cid_c = lax.axis_index("c")
        sid = lax.axis_index("s")
        idx = (idx0, idx1)
        rows = (rows0, rows1)
        sems = (sem0, sem1)

        def gchunk(j):
            return sid + j * _NS

        def valid(j):
            return gchunk(j) < n_chunks

        def issue(j, b):
            base = gchunk(j) * _CHUNK
            pltpu.async_copy(dst_hbm.at[pl.ds(e0 + base, _CHUNK)], idx[b],
                             sems[b])
            pltpu.async_copy(e_hbm.at[cid_c, pl.ds(base, _CHUNK)], rows[b],
                             sems[b])

        def wait(j, b):
            base = gchunk(j) * _CHUNK
            pltpu.make_async_copy(dst_hbm.at[pl.ds(e0 + base, _CHUNK)],
                                  idx[b], sems[b]).wait()
            pltpu.make_async_copy(e_hbm.at[cid_c, pl.ds(base, _CHUNK)],
                                  rows[b], sems[b]).wait()

        @pl.when(valid(0))
        def _():
            issue(0, 0)

        @pl.when(sid == 0)
        def _():
            pltpu.sync_copy(z_hbm.at[cid_c], acc_sh)

        plsc.subcore_barrier()

        def pair_step(i, _):
            for b in (0, 1):
                j = 2 * i + b

                @pl.when(valid(j + 1))
                def _():
                    issue(j + 1, 1 - b)

                @pl.when(valid(j))
                def _():
                    wait(j, b)
                    pltpu.sync_copy(rows[b], acc_sh.at[idx[b]], add=True)

            return _

        lax.fori_loop(0, pairs, pair_step, None)
        plsc.subcore_barrier()

        def out_step(i, _):
            ochunk = sid + i * _NS

            @pl.when(ochunk < n_out_chunks)
            def _():
                row0 = ochunk * _OUT_CHUNK
                pltpu.sync_copy(acc_sh.at[pl.ds(row0, _OUT_CHUNK)],
                                agg_hbm.at[cid_c, pl.ds(row0, _OUT_CHUNK)])

            return _

        lax.fori_loop(0, out_iters, out_step, None)

    return body(e_split, dst, init)


def _pack_bf16(y):
    h = y.shape[1] // 2
    yb = y.astype(jnp.bfloat16)
    lo = lax.bitcast_convert_type(yb[:, :h], jnp.uint16).astype(jnp.uint32)
    hi = lax.bitcast_convert_type(yb[:, h:], jnp.uint16).astype(jnp.uint32)
    return lax.bitcast_convert_type(lo | (hi << 16), jnp.int32)


def _unpack_bf16_halves(p):
    v = lax.bitcast_convert_type(p, jnp.uint32)
    lo = lax.bitcast_convert_type(v << 16, jnp.float32)
    hi = lax.bitcast_convert_type(v & jnp.uint32(0xFFFF0000), jnp.float32)
    return lo, hi


def _proj_body(n_ref, ws_ref, wd_ref, ps_ref, pd_ref):
    x = n_ref[...].astype(jnp.bfloat16)
    ps_ref[...] = _pack_bf16(
        jnp.dot(x, ws_ref[...], preferred_element_type=jnp.float32))
    pd_ref[...] = _pack_bf16(
        jnp.dot(x, wd_ref[...], preferred_element_type=jnp.float32))


def _tc_proj(n, ws, wd, bn):
    N, D = n.shape
    dh = D // 2
    grid = N // bn
    return pl.pallas_call(
        _proj_body,
        grid=(grid,),
        in_specs=[
            pl.BlockSpec((bn, D), lambda i: (i, 0)),
            pl.BlockSpec((D, D), lambda i: (0, 0)),
            pl.BlockSpec((D, D), lambda i: (0, 0)),
        ],
        out_specs=[
            pl.BlockSpec((bn, dh), lambda i: (i, 0)),
            pl.BlockSpec((bn, dh), lambda i: (i, 0)),
        ],
        out_shape=[
            jax.ShapeDtypeStruct((N, dh), jnp.int32),
            jax.ShapeDtypeStruct((N, dh), jnp.int32),
        ],
        compiler_params=pltpu.CompilerParams(
            dimension_semantics=("parallel",)),
    )(n, ws, wd)


def _mlp_ln_tail(pre, x, w2_ref, b2_ref, g_ref, beta_ref):
    h = (pre * jax.nn.sigmoid(pre)).astype(jnp.bfloat16)
    y = jnp.dot(h, w2_ref[...], preferred_element_type=jnp.float32)
    y = y + b2_ref[...]
    mu = jnp.mean(y, axis=1, keepdims=True)
    yc = y - mu
    var = jnp.mean(yc * yc, axis=1, keepdims=True)
    return yc * lax.rsqrt(var + 1e-5) * g_ref[...] + beta_ref[...] + x


def _edge_body_split(e_ref, gs_ref, gd_ref, w1_ref, b1_ref, w2_ref, b2_ref,
                     g_ref, beta_ref, *out_refs, emit_full):
    dh = gs_ref.shape[1]
    if len(e_ref.shape) == 3:
        x = jnp.concatenate([e_ref[0], e_ref[1]], axis=1)
    else:
        x = e_ref[...]
    gs_lo, gs_hi = _unpack_bf16_halves(gs_ref[...])
    gd_lo, gd_hi = _unpack_bf16_halves(gd_ref[...])
    gsum = jnp.concatenate([gs_lo + gd_lo, gs_hi + gd_hi], axis=1)
    pre = (jnp.dot(x.astype(jnp.bfloat16), w1_ref[...],
                   preferred_element_type=jnp.float32)
           + gsum + b1_ref[...])
    h = (pre * jax.nn.sigmoid(pre)).astype(jnp.bfloat16)
    y = (jnp.dot(h, w2_ref[...], preferred_element_type=jnp.float32)
         + b2_ref[...])
    mu = jnp.mean(y, axis=1, keepdims=True)
    yc = y - mu
    var = jnp.mean(yc * yc, axis=1, keepdims=True)
    out = yc * lax.rsqrt(var + 1e-5) * g_ref[...] + beta_ref[...] + x
    out_lo = out[:, :dh]
    out_hi = out[:, dh:]
    esplit_ref = out_refs[-1]
    esplit_ref[0] = out_lo
    esplit_ref[1] = out_hi
    if emit_full:
        out_refs[0][:, :dh] = out_lo
        out_refs[0][:, dh:] = out_hi


def _tc_edge_mlp(e, gs, gd, w1e, b1, w2, b2, g, beta, be, emit_full,
                 off_blocks=0):
    D = w1e.shape[0]
    dh = D // 2
    E = gs.shape[0]
    grid = E // be
    wspec = pl.BlockSpec((D, D), lambda i: (0, 0))
    vspec = pl.BlockSpec((1, D), lambda i: (0, 0))
    gspec = pl.BlockSpec((be, dh), lambda i: (i, 0))
    if e.ndim == 3:
        espec = pl.BlockSpec((2, be, dh), lambda i: (0, i, 0))
    else:
        espec = pl.BlockSpec((be, D), lambda i: (i + off_blocks, 0))
    out_specs = [pl.BlockSpec((2, be, dh), lambda i: (0, i, 0))]
    out_shape = [jax.ShapeDtypeStruct((2, E, dh), jnp.float32)]
    if emit_full:
        out_specs.insert(0, pl.BlockSpec((be, D), lambda i: (i, 0)))
        out_shape.insert(0, jax.ShapeDtypeStruct((E, D), jnp.float32))
    res = pl.pallas_call(
        functools.partial(_edge_body_split, emit_full=emit_full),
        grid=(grid,),
        in_specs=[espec, gspec, gspec, wspec, vspec, wspec, vspec, vspec,
                  vspec],
        out_specs=out_specs,
        out_shape=out_shape,
        compiler_params=pltpu.CompilerParams(
            dimension_semantics=("parallel",)),
    )(e, gs, gd, w1e, b1, w2, b2, g, beta)
    if emit_full:
        return res[0], res[1]
    return None, res[0]


def _node_body(n_ref, agg_ref, w1n_ref, w1lo_ref, w1hi_ref, b1_ref, w2_ref,
               b2_ref, g_ref, beta_ref, out_ref):
    x = n_ref[...]
    pre = (jnp.dot(x.astype(jnp.bfloat16), w1n_ref[...],
                   preferred_element_type=jnp.float32)
           + jnp.dot(agg_ref[0].astype(jnp.bfloat16), w1lo_ref[...],
                     preferred_element_type=jnp.float32)
           + jnp.dot(agg_ref[1].astype(jnp.bfloat16), w1hi_ref[...],
                     preferred_element_type=jnp.float32)
           + b1_ref[...])
    out_ref[...] = _mlp_ln_tail(pre, x, w2_ref, b2_ref, g_ref, beta_ref)


def _tc_node_mlp(n, agg, w1n, w1lo, w1hi, b1, w2, b2, g, beta, bn):
    N, D = n.shape
    dh = D // 2
    grid = N // bn
    wspec = pl.BlockSpec((D, D), lambda i: (0, 0))
    hspec = pl.BlockSpec((dh, D), lambda i: (0, 0))
    vspec = pl.BlockSpec((1, D), lambda i: (0, 0))
    return pl.pallas_call(
        _node_body,
        grid=(grid,),
        in_specs=[
            pl.BlockSpec((bn, D), lambda i: (i, 0)),
            pl.BlockSpec((2, bn, dh), lambda i: (0, i, 0)),
            wspec, hspec, hspec, vspec, wspec, vspec, vspec, vspec,
        ],
        out_specs=pl.BlockSpec((bn, D), lambda i: (i, 0)),
        out_shape=jax.ShapeDtypeStruct((N, D), jnp.float32),
        compiler_params=pltpu.CompilerParams(
            dimension_semantics=("parallel",)),
    )(n, agg, w1n, w1lo, w1hi, b1, w2, b2, g, beta)


def kernel(efeat, nfeat, src, dst, params):
    E, D = efeat.shape
    N = nfeat.shape[0]
    dh = D // 2
    be = 1600
    bn = 1000
    strip_sizes = [38400, 41600, 41600, 38400]
    strip_offs = [0, 38400, 80000, 121600]
    n_strips = len(strip_sizes)
    zeros_init = jnp.zeros((_NC, N, dh), jnp.float32)

    n = nfeat
    e_full_strips = None
    e_split_prev = [None] * n_strips
    for li, p in enumerate(params):
        last = li == len(params) - 1
        W1, b1, W2, b2, g, beta = p["edge"]
        b1r = b1.reshape(1, D)
        b2r = b2.reshape(1, D)
        gr = g.reshape(1, D)
        betar = beta.reshape(1, D)
        bf = jnp.bfloat16
        ps, pd = _tc_proj(n, W1[D:2 * D].astype(bf), W1[2 * D:].astype(bf),
                          bn)
        agg = zeros_init
        e_full_strips = []
        gathered = [
            _sc_gather(ps, pd, src, dst, strip_offs[s], strip_sizes[s])
            for s in range(n_strips)
        ]
        splits = []
        for s in range(n_strips):
            gs, gd = gathered[s]
            e_in = efeat if e_split_prev[s] is None else e_split_prev[s]
            ef, esp = _tc_edge_mlp(e_in, gs, gd, W1[:D].astype(bf), b1r,
                                   W2.astype(bf), b2r, gr, betar, be,
                                   emit_full=False,
                                   off_blocks=strip_offs[s] // be)
            splits.append(esp)
            agg = _sc_segment_sum(esp, dst, agg, strip_offs[s], N)
        e_split_prev = splits
        Wn1, nb1, Wn2, nb2, ng, nbeta = p["node"]
        n = _tc_node_mlp(n, agg, Wn1[:D].astype(bf),
                         Wn1[D:D + dh].astype(bf), Wn1[D + dh:].astype(bf),
                         nb1.reshape(1, D), Wn2.astype(bf),
                         nb2.reshape(1, D), ng.reshape(1, D),
                         nbeta.reshape(1, D), bn)
    e = jnp.concatenate(
        [jnp.concatenate([s[0], s[1]], axis=1) for s in e_split_prev],
        axis=0)
    return (e, n)


# speedup vs baseline: 1.0387x; 1.0387x over previous
import functools

import jax
import jax.numpy as jnp
from jax import lax
from jax.experimental import pallas as pl
from jax.experimental.pallas import tpu as pltpu
from jax.experimental.pallas import tpu_sc as plsc

_NC = 2
_NS = 16
_NW = _NC * _NS
_CHUNK = 128
_OUT_CHUNK = 400


def _sc_gather(ps, pd, src, dst, e0, ecount):
    D = ps.shape[1]
    dt = ps.dtype
    n_chunks = ecount // _CHUNK
    iters = (n_chunks + _NW - 1) // _NW

    mesh = plsc.VectorSubcoreMesh(core_axis_name="c", subcore_axis_name="s")

    pairs = (iters + 1) // 2

    def body(ps_hbm, pd_hbm, src_hbm, dst_hbm, gs_hbm, gd_hbm,
             idx_s0, idx_d0, buf_s0, buf_d0, idx_s1, idx_d1, buf_s1, buf_d1,
             sem_i0, sem_i1, sem_g0, sem_g1, sem_w0, sem_w1):
        idx_s = (idx_s0, idx_s1)
        idx_d = (idx_d0, idx_d1)
        buf_s = (buf_s0, buf_s1)
        buf_d = (buf_d0, buf_d1)
        sem_i = (sem_i0, sem_i1)
        sem_g = (sem_g0, sem_g1)
        sem_w = (sem_w0, sem_w1)
        wid = lax.axis_index("s") * _NC + lax.axis_index("c")

        def base_of(j):
            return (wid + j * _NW) * _CHUNK

        def valid(j):
            return wid + j * _NW < n_chunks

        def issue_idx(j, b):
            pltpu.async_copy(src_hbm.at[pl.ds(e0 + base_of(j), _CHUNK)],
                             idx_s[b], sem_i[b])
            pltpu.async_copy(dst_hbm.at[pl.ds(e0 + base_of(j), _CHUNK)],
                             idx_d[b], sem_i[b])

        def wait_idx(j, b):
            pltpu.make_async_copy(src_hbm.at[pl.ds(e0 + base_of(j), _CHUNK)],
                                  idx_s[b], sem_i[b]).wait()
            pltpu.make_async_copy(dst_hbm.at[pl.ds(e0 + base_of(j), _CHUNK)],
                                  idx_d[b], sem_i[b]).wait()

        def wait_wb(j, b):
            pltpu.make_async_copy(
                buf_s[b], gs_hbm.at[pl.ds(base_of(j), _CHUNK)],
                sem_w[b]).wait()
            pltpu.make_async_copy(
                buf_d[b], gd_hbm.at[pl.ds(base_of(j), _CHUNK)],
                sem_w[b]).wait()

        @pl.when(valid(0))
        def _():
            issue_idx(0, 0)

        def pair_step(i, _):
            for b in (0, 1):
                j = 2 * i + b

                @pl.when((j >= 2) & valid(j))
                def _():
                    wait_wb(j, b)

                @pl.when(valid(j))
                def _():
                    wait_idx(j, b)
                    pltpu.async_copy(ps_hbm.at[idx_s[b]], buf_s[b], sem_g[b])
                    pltpu.async_copy(pd_hbm.at[idx_d[b]], buf_d[b], sem_g[b])

                @pl.when(valid(j + 1))
                def _():
                    issue_idx(j + 1, 1 - b)

                @pl.when(valid(j))
                def _():
                    pltpu.make_async_copy(ps_hbm.at[idx_s[b]], buf_s[b],
                                          sem_g[b]).wait()
                    pltpu.make_async_copy(pd_hbm.at[idx_d[b]], buf_d[b],
                                          sem_g[b]).wait()
                    pltpu.async_copy(buf_s[b],
                                     gs_hbm.at[pl.ds(base_of(j), _CHUNK)],
                                     sem_w[b])
                    pltpu.async_copy(buf_d[b],
                                     gd_hbm.at[pl.ds(base_of(j), _CHUNK)],
                                     sem_w[b])

            return _

        lax.fori_loop(0, pairs, pair_step, None)
        for jt in (2 * pairs - 2, 2 * pairs - 1):
            if jt >= 0:
                bt = jt % 2

                @pl.when(valid(jt))
                def _():
                    wait_wb(jt, bt)

    scratch = [
        pltpu.VMEM((_CHUNK,), jnp.int32), pltpu.VMEM((_CHUNK,), jnp.int32),
        pltpu.VMEM((_CHUNK, D), dt), pltpu.VMEM((_CHUNK, D), dt),
        pltpu.VMEM((_CHUNK,), jnp.int32), pltpu.VMEM((_CHUNK,), jnp.int32),
        pltpu.VMEM((_CHUNK, D), dt), pltpu.VMEM((_CHUNK, D), dt),
    ] + [pltpu.SemaphoreType.DMA] * 6

    return pl.kernel(
        body,
        mesh=mesh,
        out_type=(
            jax.ShapeDtypeStruct((ecount, D), dt),
            jax.ShapeDtypeStruct((ecount, D), dt),
        ),
        scratch_types=scratch,
    )(ps, pd, src, dst)


def _sc_segment_sum(e_split, dst, init, e0, num_nodes):
    _, E, DH = e_split.shape
    n_chunks = E // _CHUNK
    iters = (n_chunks + _NS - 1) // _NS
    n_out_chunks = num_nodes // _OUT_CHUNK
    out_iters = (n_out_chunks + _NS - 1) // _NS

    pairs = (iters + 1) // 2
    mesh = plsc.VectorSubcoreMesh(core_axis_name="c", subcore_axis_name="s")

    @functools.partial(
        pl.kernel,
        mesh=mesh,
        out_type=jax.ShapeDtypeStruct((_NC, num_nodes, DH), jnp.float32),
        scratch_types=[
            pltpu.VMEM((_CHUNK,), jnp.int32),
            pltpu.VMEM((_CHUNK, DH), jnp.float32),
            pltpu.VMEM((_CHUNK,), jnp.int32),
            pltpu.VMEM((_CHUNK, DH), jnp.float32),
            pltpu.VMEM_SHARED((num_nodes, DH), jnp.float32),
            pltpu.SemaphoreType.DMA,
            pltpu.SemaphoreType.DMA,
        ],
    )
    def body(e_hbm, dst_hbm, z_hbm, agg_hbm, idx0, rows0, idx1, rows1,
             acc_sh, sem0, sem1):
        cid_c = lax.axis_index("c")
        sid = lax.axis_index("s")
        idx = (idx0, idx1)
        rows = (rows0, rows1)
        sems = (sem0, sem1)

        def gchunk(j):
            return sid + j * _NS

        def valid(j):
            return gchunk(j) < n_chunks

        def issue(j, b):
            base = gchunk(j) * _CHUNK
            pltpu.async_copy(dst_hbm.at[pl.ds(e0 + base, _CHUNK)], idx[b],
                             sems[b])
            pltpu.async_copy(e_hbm.at[cid_c, pl.ds(base, _CHUNK)], rows[b],
                             sems[b])

        def wait(j, b):
            base = gchunk(j) * _CHUNK
            pltpu.make_async_copy(dst_hbm.at[pl.ds(e0 + base, _CHUNK)],
                                  idx[b], sems[b]).wait()
            pltpu.make_async_copy(e_hbm.at[cid_c, pl.ds(base, _CHUNK)],
                                  rows[b], sems[b]).wait()

        @pl.when(valid(0))
        def _():
            issue(0, 0)

        @pl.when(sid == 0)
        def _():
            pltpu.sync_copy(z_hbm.at[cid_c], acc_sh)

        plsc.subcore_barrier()

        def pair_step(i, _):
            for b in (0, 1):
                j = 2 * i + b

                @pl.when(valid(j + 1))
                def _():
                    issue(j + 1, 1 - b)

                @pl.when(valid(j))
                def _():
                    wait(j, b)
                    pltpu.sync_copy(rows[b], acc_sh.at[idx[b]], add=True)

            return _

        lax.fori_loop(0, pairs, pair_step, None)
        plsc.subcore_barrier()

        def out_step(i, _):
            ochunk = sid + i * _NS

            @pl.when(ochunk < n_out_chunks)
            def _():
                row0 = ochunk * _OUT_CHUNK
                pltpu.sync_copy(acc_sh.at[pl.ds(row0, _OUT_CHUNK)],
                                agg_hbm.at[cid_c, pl.ds(row0, _OUT_CHUNK)])

            return _

        lax.fori_loop(0, out_iters, out_step, None)

    return body(e_split, dst, init)


def _pack_bf16(y):
    h = y.shape[1] // 2
    yb = y.astype(jnp.bfloat16)
    lo = lax.bitcast_convert_type(yb[:, :h], jnp.uint16).astype(jnp.uint32)
    hi = lax.bitcast_convert_type(yb[:, h:], jnp.uint16).astype(jnp.uint32)
    return lax.bitcast_convert_type(lo | (hi << 16), jnp.int32)


def _unpack_bf16_halves(p):
    v = lax.bitcast_convert_type(p, jnp.uint32)
    lo = lax.bitcast_convert_type(v << 16, jnp.float32)
    hi = lax.bitcast_convert_type(v & jnp.uint32(0xFFFF0000), jnp.float32)
    return lo, hi


def _proj_body(n_ref, ws_ref, wd_ref, ps_ref, pd_ref):
    x = n_ref[...].astype(jnp.bfloat16)
    ps_ref[...] = _pack_bf16(
        jnp.dot(x, ws_ref[...], preferred_element_type=jnp.float32))
    pd_ref[...] = _pack_bf16(
        jnp.dot(x, wd_ref[...], preferred_element_type=jnp.float32))


def _tc_proj(n, ws, wd, bn):
    N, D = n.shape
    dh = D // 2
    grid = N // bn
    return pl.pallas_call(
        _proj_body,
        grid=(grid,),
        in_specs=[
            pl.BlockSpec((bn, D), lambda i: (i, 0)),
            pl.BlockSpec((D, D), lambda i: (0, 0)),
            pl.BlockSpec((D, D), lambda i: (0, 0)),
        ],
        out_specs=[
            pl.BlockSpec((bn, dh), lambda i: (i, 0)),
            pl.BlockSpec((bn, dh), lambda i: (i, 0)),
        ],
        out_shape=[
            jax.ShapeDtypeStruct((N, dh), jnp.int32),
            jax.ShapeDtypeStruct((N, dh), jnp.int32),
        ],
        compiler_params=pltpu.CompilerParams(
            dimension_semantics=("parallel",)),
    )(n, ws, wd)


def _mlp_ln_tail(pre, x, w2_ref, b2_ref, g_ref, beta_ref):
    h = (pre * jax.nn.sigmoid(pre)).astype(jnp.bfloat16)
    y = jnp.dot(h, w2_ref[...], preferred_element_type=jnp.float32)
    y = y + b2_ref[...]
    mu = jnp.mean(y, axis=1, keepdims=True)
    yc = y - mu
    var = jnp.mean(yc * yc, axis=1, keepdims=True)
    return yc * lax.rsqrt(var + 1e-5) * g_ref[...] + beta_ref[...] + x


def _edge_body_split(e_ref, gs_ref, gd_ref, w1_ref, b1_ref, w2_ref, b2_ref,
                     g_ref, beta_ref, *out_refs, emit_full):
    dh = gs_ref.shape[1]
    if len(e_ref.shape) == 3:
        x = jnp.concatenate([e_ref[0], e_ref[1]], axis=1)
    else:
        x = e_ref[...]
    gs_lo, gs_hi = _unpack_bf16_halves(gs_ref[...])
    gd_lo, gd_hi = _unpack_bf16_halves(gd_ref[...])
    gsum = jnp.concatenate([gs_lo + gd_lo, gs_hi + gd_hi], axis=1)
    pre = (jnp.dot(x.astype(jnp.bfloat16), w1_ref[...],
                   preferred_element_type=jnp.float32)
           + gsum + b1_ref[...])
    h = (pre * jax.nn.sigmoid(pre)).astype(jnp.bfloat16)
    y = (jnp.dot(h, w2_ref[...], preferred_element_type=jnp.float32)
         + b2_ref[...])
    mu = jnp.mean(y, axis=1, keepdims=True)
    yc = y - mu
    var = jnp.mean(yc * yc, axis=1, keepdims=True)
    out = yc * lax.rsqrt(var + 1e-5) * g_ref[...] + beta_ref[...] + x
    out_lo = out[:, :dh]
    out_hi = out[:, dh:]
    esplit_ref = out_refs[-1]
    esplit_ref[0] = out_lo
    esplit_ref[1] = out_hi
    if emit_full:
        out_refs[0][:, :dh] = out_lo
        out_refs[0][:, dh:] = out_hi


def _tc_edge_mlp(e, gs, gd, w1e, b1, w2, b2, g, beta, be, emit_full,
                 off_blocks=0):
    D = w1e.shape[0]
    dh = D // 2
    E = gs.shape[0]
    grid = E // be
    wspec = pl.BlockSpec((D, D), lambda i: (0, 0))
    vspec = pl.BlockSpec((1, D), lambda i: (0, 0))
    gspec = pl.BlockSpec((be, dh), lambda i: (i, 0))
    if e.ndim == 3:
        espec = pl.BlockSpec((2, be, dh), lambda i: (0, i, 0))
    else:
        espec = pl.BlockSpec((be, D), lambda i: (i + off_blocks, 0))
    out_specs = [pl.BlockSpec((2, be, dh), lambda i: (0, i, 0))]
    out_shape = [jax.ShapeDtypeStruct((2, E, dh), jnp.float32)]
    if emit_full:
        out_specs.insert(0, pl.BlockSpec((be, D), lambda i: (i, 0)))
        out_shape.insert(0, jax.ShapeDtypeStruct((E, D), jnp.float32))
    res = pl.pallas_call(
        functools.partial(_edge_body_split, emit_full=emit_full),
        grid=(grid,),
        in_specs=[espec, gspec, gspec, wspec, vspec, wspec, vspec, vspec,
                  vspec],
        out_specs=out_specs,
        out_shape=out_shape,
        compiler_params=pltpu.CompilerParams(
            dimension_semantics=("parallel",)),
    )(e, gs, gd, w1e, b1, w2, b2, g, beta)
    if emit_full:
        return res[0], res[1]
    return None, res[0]


def _node_body(n_ref, agg_ref, w1n_ref, w1lo_ref, w1hi_ref, b1_ref, w2_ref,
               b2_ref, g_ref, beta_ref, out_ref):
    x = n_ref[...]
    pre = (jnp.dot(x.astype(jnp.bfloat16), w1n_ref[...],
                   preferred_element_type=jnp.float32)
           + jnp.dot(agg_ref[0].astype(jnp.bfloat16), w1lo_ref[...],
                     preferred_element_type=jnp.float32)
           + jnp.dot(agg_ref[1].astype(jnp.bfloat16), w1hi_ref[...],
                     preferred_element_type=jnp.float32)
           + b1_ref[...])
    out_ref[...] = _mlp_ln_tail(pre, x, w2_ref, b2_ref, g_ref, beta_ref)


def _tc_node_mlp(n, agg, w1n, w1lo, w1hi, b1, w2, b2, g, beta, bn):
    N, D = n.shape
    dh = D // 2
    grid = N // bn
    wspec = pl.BlockSpec((D, D), lambda i: (0, 0))
    hspec = pl.BlockSpec((dh, D), lambda i: (0, 0))
    vspec = pl.BlockSpec((1, D), lambda i: (0, 0))
    return pl.pallas_call(
        _node_body,
        grid=(grid,),
        in_specs=[
            pl.BlockSpec((bn, D), lambda i: (i, 0)),
            pl.BlockSpec((2, bn, dh), lambda i: (0, i, 0)),
            wspec, hspec, hspec, vspec, wspec, vspec, vspec, vspec,
        ],
        out_specs=pl.BlockSpec((bn, D), lambda i: (i, 0)),
        out_shape=jax.ShapeDtypeStruct((N, D), jnp.float32),
        compiler_params=pltpu.CompilerParams(
            dimension_semantics=("parallel",)),
    )(n, agg, w1n, w1lo, w1hi, b1, w2, b2, g, beta)


def kernel(efeat, nfeat, src, dst, params):
    E, D = efeat.shape
    N = nfeat.shape[0]
    dh = D // 2
    be = 1600
    bn = 1000
    strip_sizes = [38400, 41600, 41600, 38400]
    strip_offs = [0, 38400, 80000, 121600]
    n_strips = len(strip_sizes)
    zeros_init = jnp.zeros((_NC, N, dh), jnp.float32)

    n = nfeat
    e_full_strips = None
    e_split_prev = [None] * n_strips
    for li, p in enumerate(params):
        last = li == len(params) - 1
        W1, b1, W2, b2, g, beta = p["edge"]
        b1r = b1.reshape(1, D)
        b2r = b2.reshape(1, D)
        gr = g.reshape(1, D)
        betar = beta.reshape(1, D)
        bf = jnp.bfloat16
        ps, pd = _tc_proj(n, W1[D:2 * D].astype(bf), W1[2 * D:].astype(bf),
                          bn)
        agg = zeros_init
        e_full_strips = []
        gathered = [
            _sc_gather(ps, pd, src, dst, strip_offs[s], strip_sizes[s])
            for s in range(n_strips)
        ]
        splits = []
        for s in range(n_strips):
            gs, gd = gathered[s]
            e_in = efeat if e_split_prev[s] is None else e_split_prev[s]
            bes = 800 if last else be
            ef, esp = _tc_edge_mlp(e_in, gs, gd, W1[:D].astype(bf), b1r,
                                   W2.astype(bf), b2r, gr, betar, bes,
                                   emit_full=last,
                                   off_blocks=strip_offs[s] // bes)
            e_full_strips.append(ef)
            splits.append(esp)
            agg = _sc_segment_sum(esp, dst, agg, strip_offs[s], N)
        e_split_prev = splits
        Wn1, nb1, Wn2, nb2, ng, nbeta = p["node"]
        n = _tc_node_mlp(n, agg, Wn1[:D].astype(bf),
                         Wn1[D:D + dh].astype(bf), Wn1[D + dh:].astype(bf),
                         nb1.reshape(1, D), Wn2.astype(bf),
                         nb2.reshape(1, D), ng.reshape(1, D),
                         nbeta.reshape(1, D), bn)
    e = jnp.concatenate(e_full_strips, axis=0)
    return (e, n)


# speedup vs baseline: 1.0875x; 1.0470x over previous
import functools

import jax
import jax.numpy as jnp
from jax import lax
from jax.experimental import pallas as pl
from jax.experimental.pallas import tpu as pltpu
from jax.experimental.pallas import tpu_sc as plsc

_NC = 2
_NS = 16
_NW = _NC * _NS
_CHUNK = 128
_OUT_CHUNK = 400


def _sc_gather(ps, pd, src, dst, e0, ecount):
    D = ps.shape[1]
    dt = ps.dtype
    n_chunks = ecount // _CHUNK
    iters = (n_chunks + _NW - 1) // _NW

    mesh = plsc.VectorSubcoreMesh(core_axis_name="c", subcore_axis_name="s")

    pairs = (iters + 1) // 2

    def body(ps_hbm, pd_hbm, src_hbm, dst_hbm, gs_hbm, gd_hbm,
             idx_s0, idx_d0, buf_s0, buf_d0, idx_s1, idx_d1, buf_s1, buf_d1,
             sem_i0, sem_i1, sem_g0, sem_g1, sem_w0, sem_w1):
        idx_s = (idx_s0, idx_s1)
        idx_d = (idx_d0, idx_d1)
        buf_s = (buf_s0, buf_s1)
        buf_d = (buf_d0, buf_d1)
        sem_i = (sem_i0, sem_i1)
        sem_g = (sem_g0, sem_g1)
        sem_w = (sem_w0, sem_w1)
        wid = lax.axis_index("s") * _NC + lax.axis_index("c")

        def base_of(j):
            return (wid + j * _NW) * _CHUNK

        def valid(j):
            return wid + j * _NW < n_chunks

        def issue_idx(j, b):
            pltpu.async_copy(src_hbm.at[pl.ds(e0 + base_of(j), _CHUNK)],
                             idx_s[b], sem_i[b])
            pltpu.async_copy(dst_hbm.at[pl.ds(e0 + base_of(j), _CHUNK)],
                             idx_d[b], sem_i[b])

        def wait_idx(j, b):
            pltpu.make_async_copy(src_hbm.at[pl.ds(e0 + base_of(j), _CHUNK)],
                                  idx_s[b], sem_i[b]).wait()
            pltpu.make_async_copy(dst_hbm.at[pl.ds(e0 + base_of(j), _CHUNK)],
                                  idx_d[b], sem_i[b]).wait()

        def wait_wb(j, b):
            pltpu.make_async_copy(
                buf_s[b], gs_hbm.at[pl.ds(base_of(j), _CHUNK)],
                sem_w[b]).wait()
            pltpu.make_async_copy(
                buf_d[b], gd_hbm.at[pl.ds(base_of(j), _CHUNK)],
                sem_w[b]).wait()

        @pl.when(valid(0))
        def _():
            issue_idx(0, 0)

        def pair_step(i, _):
            for b in (0, 1):
                j = 2 * i + b

                @pl.when((j >= 2) & valid(j))
                def _():
                    wait_wb(j, b)

                @pl.when(valid(j))
                def _():
                    wait_idx(j, b)
                    pltpu.async_copy(ps_hbm.at[idx_s[b]], buf_s[b], sem_g[b])
                    pltpu.async_copy(pd_hbm.at[idx_d[b]], buf_d[b], sem_g[b])

                @pl.when(valid(j + 1))
                def _():
                    issue_idx(j + 1, 1 - b)

                @pl.when(valid(j))
                def _():
                    pltpu.make_async_copy(ps_hbm.at[idx_s[b]], buf_s[b],
                                          sem_g[b]).wait()
                    pltpu.make_async_copy(pd_hbm.at[idx_d[b]], buf_d[b],
                                          sem_g[b]).wait()
                    pltpu.async_copy(buf_s[b],
                                     gs_hbm.at[pl.ds(base_of(j), _CHUNK)],
                                     sem_w[b])
                    pltpu.async_copy(buf_d[b],
                                     gd_hbm.at[pl.ds(base_of(j), _CHUNK)],
                                     sem_w[b])

            return _

        lax.fori_loop(0, pairs, pair_step, None)
        for jt in (2 * pairs - 2, 2 * pairs - 1):
            if jt >= 0:
                bt = jt % 2

                @pl.when(valid(jt))
                def _():
                    wait_wb(jt, bt)

    scratch = [
        pltpu.VMEM((_CHUNK,), jnp.int32), pltpu.VMEM((_CHUNK,), jnp.int32),
        pltpu.VMEM((_CHUNK, D), dt), pltpu.VMEM((_CHUNK, D), dt),
        pltpu.VMEM((_CHUNK,), jnp.int32), pltpu.VMEM((_CHUNK,), jnp.int32),
        pltpu.VMEM((_CHUNK, D), dt), pltpu.VMEM((_CHUNK, D), dt),
    ] + [pltpu.SemaphoreType.DMA] * 6

    return pl.kernel(
        body,
        mesh=mesh,
        out_type=(
            jax.ShapeDtypeStruct((ecount, D), dt),
            jax.ShapeDtypeStruct((ecount, D), dt),
        ),
        scratch_types=scratch,
    )(ps, pd, src, dst)


def _sc_segment_sum(e_split, dst, init, e0, num_nodes):
    _, E, DH = e_split.shape
    n_chunks = E // _CHUNK
    iters = (n_chunks + _NS - 1) // _NS
    n_out_chunks = num_nodes // _OUT_CHUNK
    out_iters = (n_out_chunks + _NS - 1) // _NS

    pairs = (iters + 1) // 2
    mesh = plsc.VectorSubcoreMesh(core_axis_name="c", subcore_axis_name="s")

    @functools.partial(
        pl.kernel,
        mesh=mesh,
        out_type=jax.ShapeDtypeStruct((_NC, num_nodes, DH), jnp.float32),
        scratch_types=[
            pltpu.VMEM((_CHUNK,), jnp.int32),
            pltpu.VMEM((_CHUNK, DH), jnp.float32),
            pltpu.VMEM((_CHUNK,), jnp.int32),
            pltpu.VMEM((_CHUNK, DH), jnp.float32),
            pltpu.VMEM_SHARED((num_nodes, DH), jnp.float32),
            pltpu.SemaphoreType.DMA,
            pltpu.SemaphoreType.DMA,
        ],
    )
    def body(e_hbm, dst_hbm, z_hbm, agg_hbm, idx0, rows0, idx1, rows1,
             acc_sh, sem0, sem1):
        cid_c = lax.axis_index("c")
        sid = lax.axis_index("s")
        idx = (idx0, idx1)
        rows = (rows0, rows1)
        sems = (sem0, sem1)

        def gchunk(j):
            return sid + j * _NS

        def valid(j):
            return gchunk(j) < n_chunks

        def issue(j, b):
            base = gchunk(j) * _CHUNK
            pltpu.async_copy(dst_hbm.at[pl.ds(e0 + base, _CHUNK)], idx[b],
                             sems[b])
            pltpu.async_copy(e_hbm.at[cid_c, pl.ds(base, _CHUNK)], rows[b],
                             sems[b])

        def wait(j, b):
            base = gchunk(j) * _CHUNK
            pltpu.make_async_copy(dst_hbm.at[pl.ds(e0 + base, _CHUNK)],
                                  idx[b], sems[b]).wait()
            pltpu.make_async_copy(e_hbm.at[cid_c, pl.ds(base, _CHUNK)],
                                  rows[b], sems[b]).wait()

        @pl.when(valid(0))
        def _():
            issue(0, 0)

        @pl.when(sid == 0)
        def _():
            pltpu.sync_copy(z_hbm.at[cid_c], acc_sh)

        plsc.subcore_barrier()

        def pair_step(i, _):
            for b in (0, 1):
                j = 2 * i + b

                @pl.when(valid(j + 1))
                def _():
                    issue(j + 1, 1 - b)

                @pl.when(valid(j))
                def _():
                    wait(j, b)
                    pltpu.sync_copy(rows[b], acc_sh.at[idx[b]], add=True)

            return _

        lax.fori_loop(0, pairs, pair_step, None)
        plsc.subcore_barrier()

        def out_step(i, _):
            ochunk = sid + i * _NS

            @pl.when(ochunk < n_out_chunks)
            def _():
                row0 = ochunk * _OUT_CHUNK
                pltpu.sync_copy(acc_sh.at[pl.ds(row0, _OUT_CHUNK)],
                                agg_hbm.at[cid_c, pl.ds(row0, _OUT_CHUNK)])

            return _

        lax.fori_loop(0, out_iters, out_step, None)

    return body(e_split, dst, init)


def _pack_bf16(y):
    h = y.shape[1] // 2
    yb = y.astype(jnp.bfloat16)
    lo = lax.bitcast_convert_type(yb[:, :h], jnp.uint16).astype(jnp.uint32)
    hi = lax.bitcast_convert_type(yb[:, h:], jnp.uint16).astype(jnp.uint32)
    return lax.bitcast_convert_type(lo | (hi << 16), jnp.int32)


def _unpack_bf16_halves(p):
    v = lax.bitcast_convert_type(p, jnp.uint32)
    lo = lax.bitcast_convert_type(v << 16, jnp.float32)
    hi = lax.bitcast_convert_type(v & jnp.uint32(0xFFFF0000), jnp.float32)
    return lo, hi


def _proj_body(n_ref, ws_ref, wd_ref, ps_ref, pd_ref):
    x = n_ref[...].astype(jnp.bfloat16)
    ps_ref[...] = _pack_bf16(
        jnp.dot(x, ws_ref[...], preferred_element_type=jnp.float32))
    pd_ref[...] = _pack_bf16(
        jnp.dot(x, wd_ref[...], preferred_element_type=jnp.float32))


def _tc_proj(n, ws, wd, bn):
    N, D = n.shape
    dh = D // 2
    grid = N // bn
    return pl.pallas_call(
        _proj_body,
        grid=(grid,),
        in_specs=[
            pl.BlockSpec((bn, D), lambda i: (i, 0)),
            pl.BlockSpec((D, D), lambda i: (0, 0)),
            pl.BlockSpec((D, D), lambda i: (0, 0)),
        ],
        out_specs=[
            pl.BlockSpec((bn, dh), lambda i: (i, 0)),
            pl.BlockSpec((bn, dh), lambda i: (i, 0)),
        ],
        out_shape=[
            jax.ShapeDtypeStruct((N, dh), jnp.int32),
            jax.ShapeDtypeStruct((N, dh), jnp.int32),
        ],
        compiler_params=pltpu.CompilerParams(
            dimension_semantics=("parallel",)),
    )(n, ws, wd)


def _mlp_ln_tail(pre, x, w2_ref, b2_ref, g_ref, beta_ref):
    h = (pre * jax.nn.sigmoid(pre)).astype(jnp.bfloat16)
    y = jnp.dot(h, w2_ref[...], preferred_element_type=jnp.float32)
    y = y + b2_ref[...]
    mu = jnp.mean(y, axis=1, keepdims=True)
    yc = y - mu
    var = jnp.mean(yc * yc, axis=1, keepdims=True)
    return yc * lax.rsqrt(var + 1e-5) * g_ref[...] + beta_ref[...] + x


def _edge_body_split(e_ref, gs_ref, gd_ref, w1_ref, b1_ref, w2_ref, b2_ref,
                     g_ref, beta_ref, *out_refs, emit_full):
    dh = gs_ref.shape[1]
    if len(e_ref.shape) == 3:
        x = jnp.concatenate([e_ref[0], e_ref[1]], axis=1)
    else:
        x = e_ref[...]
    gs_lo, gs_hi = _unpack_bf16_halves(gs_ref[...])
    gd_lo, gd_hi = _unpack_bf16_halves(gd_ref[...])
    gsum = jnp.concatenate([gs_lo + gd_lo, gs_hi + gd_hi], axis=1)
    pre = (jnp.dot(x.astype(jnp.bfloat16), w1_ref[...],
                   preferred_element_type=jnp.float32)
           + gsum + b1_ref[...])
    h = (pre * jax.nn.sigmoid(pre)).astype(jnp.bfloat16)
    y = (jnp.dot(h, w2_ref[...], preferred_element_type=jnp.float32)
         + b2_ref[...])
    mu = jnp.mean(y, axis=1, keepdims=True)
    yc = y - mu
    var = jnp.mean(yc * yc, axis=1, keepdims=True)
    out = yc * lax.rsqrt(var + 1e-5) * g_ref[...] + beta_ref[...] + x
    out_lo = out[:, :dh]
    out_hi = out[:, dh:]
    esplit_ref = out_refs[-1]
    esplit_ref[0] = out_lo
    esplit_ref[1] = out_hi
    if emit_full:
        out_refs[1][:, :dh] = out_lo
        out_refs[1][:, dh:] = out_hi


def _tc_edge_mlp(e, gs, gd, w1e, b1, w2, b2, g, beta, be, emit_full,
                 off_blocks=0, e_acc=None):
    D = w1e.shape[0]
    dh = D // 2
    E = gs.shape[0]
    grid = E // be
    wspec = pl.BlockSpec((D, D), lambda i: (0, 0))
    vspec = pl.BlockSpec((1, D), lambda i: (0, 0))
    gspec = pl.BlockSpec((be, dh), lambda i: (i, 0))
    if e.ndim == 3:
        espec = pl.BlockSpec((2, be, dh), lambda i: (0, i, 0))
    else:
        espec = pl.BlockSpec((be, D), lambda i: (i + off_blocks, 0))
    out_specs = [pl.BlockSpec((2, be, dh), lambda i: (0, i, 0))]
    out_shape = [jax.ShapeDtypeStruct((2, E, dh), jnp.float32)]
    in_specs = [espec, gspec, gspec, wspec, vspec, wspec, vspec, vspec,
                vspec]
    args = [e, gs, gd, w1e, b1, w2, b2, g, beta]
    io_aliases = {}
    if emit_full:
        in_specs.append(pl.BlockSpec(memory_space=pltpu.MemorySpace.HBM))
        args.append(e_acc)
        out_specs.insert(0,
                         pl.BlockSpec((be, D), lambda i: (i + off_blocks, 0)))
        out_shape.insert(0,
                         jax.ShapeDtypeStruct(e_acc.shape, jnp.float32))
        io_aliases = {9: 0}
    res = pl.pallas_call(
        functools.partial(_edge_body_split, emit_full=emit_full),
        grid=(grid,),
        in_specs=in_specs,
        out_specs=out_specs,
        out_shape=out_shape,
        input_output_aliases=io_aliases,
        compiler_params=pltpu.CompilerParams(
            dimension_semantics=("parallel",)),
    )(*args)
    if emit_full:
        return res[0], res[1]
    return None, res[0]


def _node_body(n_ref, agg_ref, w1n_ref, w1lo_ref, w1hi_ref, b1_ref, w2_ref,
               b2_ref, g_ref, beta_ref, out_ref):
    x = n_ref[...]
    pre = (jnp.dot(x.astype(jnp.bfloat16), w1n_ref[...],
                   preferred_element_type=jnp.float32)
           + jnp.dot(agg_ref[0].astype(jnp.bfloat16), w1lo_ref[...],
                     preferred_element_type=jnp.float32)
           + jnp.dot(agg_ref[1].astype(jnp.bfloat16), w1hi_ref[...],
                     preferred_element_type=jnp.float32)
           + b1_ref[...])
    out_ref[...] = _mlp_ln_tail(pre, x, w2_ref, b2_ref, g_ref, beta_ref)


def _tc_node_mlp(n, agg, w1n, w1lo, w1hi, b1, w2, b2, g, beta, bn):
    N, D = n.shape
    dh = D // 2
    grid = N // bn
    wspec = pl.BlockSpec((D, D), lambda i: (0, 0))
    hspec = pl.BlockSpec((dh, D), lambda i: (0, 0))
    vspec = pl.BlockSpec((1, D), lambda i: (0, 0))
    return pl.pallas_call(
        _node_body,
        grid=(grid,),
        in_specs=[
            pl.BlockSpec((bn, D), lambda i: (i, 0)),
            pl.BlockSpec((2, bn, dh), lambda i: (0, i, 0)),
            wspec, hspec, hspec, vspec, wspec, vspec, vspec, vspec,
        ],
        out_specs=pl.BlockSpec((bn, D), lambda i: (i, 0)),
        out_shape=jax.ShapeDtypeStruct((N, D), jnp.float32),
        compiler_params=pltpu.CompilerParams(
            dimension_semantics=("parallel",)),
    )(n, agg, w1n, w1lo, w1hi, b1, w2, b2, g, beta)


def kernel(efeat, nfeat, src, dst, params):
    E, D = efeat.shape
    N = nfeat.shape[0]
    dh = D // 2
    be = 1600
    bn = 1000
    strip_sizes = [38400, 41600, 41600, 38400]
    strip_offs = [0, 38400, 80000, 121600]
    n_strips = len(strip_sizes)
    zeros_init = jnp.zeros((_NC, N, dh), jnp.float32)

    n = nfeat
    e_full_strips = None
    e_split_prev = [None] * n_strips
    for li, p in enumerate(params):
        last = li == len(params) - 1
        W1, b1, W2, b2, g, beta = p["edge"]
        b1r = b1.reshape(1, D)
        b2r = b2.reshape(1, D)
        gr = g.reshape(1, D)
        betar = beta.reshape(1, D)
        bf = jnp.bfloat16
        ps, pd = _tc_proj(n, W1[D:2 * D].astype(bf), W1[2 * D:].astype(bf),
                          bn)
        agg = zeros_init
        e_acc = jnp.zeros((E, D), jnp.float32) if last else None
        gathered = [
            _sc_gather(ps, pd, src, dst, strip_offs[s], strip_sizes[s])
            for s in range(n_strips)
        ]
        splits = []
        for s in range(n_strips):
            gs, gd = gathered[s]
            e_in = efeat if e_split_prev[s] is None else e_split_prev[s]
            bes = 800 if last else be
            ef, esp = _tc_edge_mlp(e_in, gs, gd, W1[:D].astype(bf), b1r,
                                   W2.astype(bf), b2r, gr, betar, bes,
                                   emit_full=last,
                                   off_blocks=strip_offs[s] // bes,
                                   e_acc=e_acc)
            if last:
                e_acc = ef
            splits.append(esp)
            agg = _sc_segment_sum(esp, dst, agg, strip_offs[s], N)
        e_split_prev = splits
        Wn1, nb1, Wn2, nb2, ng, nbeta = p["node"]
        n = _tc_node_mlp(n, agg, Wn1[:D].astype(bf),
                         Wn1[D:D + dh].astype(bf), Wn1[D + dh:].astype(bf),
                         nb1.reshape(1, D), Wn2.astype(bf),
                         nb2.reshape(1, D), ng.reshape(1, D),
                         nbeta.reshape(1, D), bn)
    return (e_acc, n)
